# baseline (speedup 1.0000x reference)
import functools

import jax
import jax.numpy as jnp
from jax import lax
from jax.experimental import pallas as pl
from jax.experimental.pallas import tpu as pltpu
from jax.experimental.pallas import tpu_sc as plsc

NUM_SEGMENTS = 64
N_COLS = 512

TC_ROWS = 82080
TC_BLOCK = 4560
SC_ROWS = 17920
SC_WORKERS = 32
SC_RPW = SC_ROWS // SC_WORKERS
SC_CHUNK = 56
SC_NCHUNK = SC_RPW // SC_CHUNK
N_ACC = 8


def _tc_body(p_ref, t_ref, o_ref):
    i = pl.program_id(0)

    @pl.when(i == 0)
    def _init():
        o_ref[0, 0] = 0.0

    o_ref[0, 0] += jnp.sum(jnp.abs(p_ref[...] - t_ref[...]))


def _tc_partial(preds, target):
    return pl.pallas_call(
        _tc_body,
        grid=(TC_ROWS // TC_BLOCK,),
        in_specs=[
            pl.BlockSpec((TC_BLOCK, N_COLS), lambda i: (i, 0)),
            pl.BlockSpec((TC_BLOCK, N_COLS), lambda i: (i, 0)),
        ],
        out_specs=pl.BlockSpec(
            (1, 1), lambda i: (0, 0), memory_space=pltpu.SMEM
        ),
        out_shape=jax.ShapeDtypeStruct((1, 1), jnp.float32),
        compiler_params=pltpu.CompilerParams(
            dimension_semantics=("arbitrary",),
        ),
    )(preds, target)


@functools.partial(
    pl.kernel,
    mesh=plsc.VectorSubcoreMesh(core_axis_name="c", subcore_axis_name="s"),
    out_type=jax.ShapeDtypeStruct((SC_WORKERS, 16), jnp.float32),
    scratch_types=[
        pltpu.VMEM((2, SC_CHUNK, N_COLS), jnp.float32),
        pltpu.VMEM((2, SC_CHUNK, N_COLS), jnp.float32),
        pltpu.VMEM((16,), jnp.float32),
        pltpu.SemaphoreType.DMA,
        pltpu.SemaphoreType.DMA,
    ],
)
def _sc_partial(p_hbm, t_hbm, out_hbm, p_v, t_v, acc_v, sem0, sem1):
    sems = (sem0, sem1)
    wid = lax.axis_index("s") * 2 + lax.axis_index("c")
    base = TC_ROWS + wid * SC_RPW

    def fire(ci):
        slot = ci % 2
        r0 = base + ci * SC_CHUNK
        hp = pltpu.async_copy(
            p_hbm.at[pl.ds(r0, SC_CHUNK)], p_v.at[slot], sems[slot])
        ht = pltpu.async_copy(
            t_hbm.at[pl.ds(r0, SC_CHUNK)], t_v.at[slot], sems[slot])
        return hp, ht

    accs = tuple(jnp.zeros((16,), jnp.float32) for _ in range(N_ACC))
    pending = fire(0)
    for ci in range(SC_NCHUNK):
        slot = ci % 2
        nxt = fire(ci + 1) if ci + 1 < SC_NCHUNK else None
        pending[0].wait()
        pending[1].wait()
        pending = nxt

        def row_body(r, accs):
            accs = list(accs)
            for c in range(N_COLS // 16):
                pv = p_v[slot, r, pl.ds(c * 16, 16)]
                tv = t_v[slot, r, pl.ds(c * 16, 16)]
                accs[c % N_ACC] = accs[c % N_ACC] + jnp.abs(pv - tv)
            return tuple(accs)

        accs = lax.fori_loop(0, SC_CHUNK, row_body, accs)

    total = accs[0]
    for a in accs[1:]:
        total = total + a
    acc_v[...] = total
    pltpu.sync_copy(acc_v, out_hbm.at[wid])


def kernel(preds, target, batch_map):
    tc_sum = _tc_partial(preds, target)
    sc_partials = _sc_partial(preds, target)
    total = tc_sum[0, 0] + jnp.sum(sc_partials)
    return total / (NUM_SEGMENTS * 512.0)


# speedup vs baseline: 1.1946x; 1.1946x over previous
import jax
import jax.numpy as jnp
from jax.experimental import pallas as pl
from jax.experimental.pallas import tpu as pltpu

NUM_SEGMENTS = 64
ROW_BLOCK = 4000
COL_BLOCK = 256


def _reduce_body(pl_ref, pr_ref, tl_ref, tr_ref, o_ref):
    i = pl.program_id(0)

    @pl.when(i == 0)
    def _init():
        o_ref[0, 0] = 0.0

    s = (jnp.sum(jnp.abs(pl_ref[...] - tl_ref[...]))
         + jnp.sum(jnp.abs(pr_ref[...] - tr_ref[...])))
    o_ref[0, 0] += s

    @pl.when(i == pl.num_programs(0) - 1)
    def _finalize():
        o_ref[0, 0] = o_ref[0, 0] / (NUM_SEGMENTS * 512.0)


def kernel(preds, target, batch_map):
    n_rows, n_cols = preds.shape
    grid = (n_rows // ROW_BLOCK,)
    half = pl.BlockSpec((ROW_BLOCK, COL_BLOCK), lambda i: (i, 0))
    half_r = pl.BlockSpec((ROW_BLOCK, COL_BLOCK), lambda i: (i, 1))
    out = pl.pallas_call(
        _reduce_body,
        grid=grid,
        in_specs=[half, half_r, half, half_r],
        out_specs=pl.BlockSpec(
            (1, 1), lambda i: (0, 0), memory_space=pltpu.SMEM
        ),
        out_shape=jax.ShapeDtypeStruct((1, 1), jnp.float32),
        compiler_params=pltpu.CompilerParams(
            dimension_semantics=("arbitrary",),
        ),
    )(preds, preds, target, target)
    return out[0, 0]
